# Initial kernel scaffold; baseline (speedup 1.0000x reference)
import jax
import jax.numpy as jnp
from jax.experimental import pallas as pl


def kernel(hidden_states, enc_w1, enc_b1, enc_g, enc_beta, enc_w2, enc_b2, dec_w1, dec_b1, dec_g, dec_beta, dec_w2, dec_b2, bin_edges):
    raise NotImplementedError("write your pallas kernel here")


# speedup vs baseline: 1.1098x; 1.1098x over previous
import functools
import math

import jax
import jax.numpy as jnp
from jax.experimental import pallas as pl
from jax.experimental.pallas import tpu as pltpu

_EPS = 1e-5


def _layernorm(x, g, b):
    mu = x.mean(axis=-1, keepdims=True)
    var = ((x - mu) ** 2).mean(axis=-1, keepdims=True)
    return (x - mu) * jax.lax.rsqrt(var + _EPS) * g + b


def _gelu_exact(x):
    return 0.5 * x * (1.0 + jax.lax.erf(x * (1.0 / math.sqrt(2.0))))


def _pool_mlp_kernel(n_chunks, n_edges,
                     x_ref, ew1_ref, eb1_ref, eg_ref, ebeta_ref, ew2_ref,
                     eb2_ref, dw1_ref, db1_ref, dg_ref, dbeta_ref, dw2_ref,
                     db2_ref, edges_ref, out_ref, acc_ref):
    j = pl.program_id(1)

    @pl.when(j == 0)
    def _():
        acc_ref[...] = jnp.zeros_like(acc_ref)

    acc_ref[...] += jnp.sum(x_ref[0], axis=0, keepdims=True)

    @pl.when(j == n_chunks - 1)
    def _():
        seq = x_ref.shape[1] * n_chunks
        pooled = acc_ref[...] * (1.0 / seq)
        h = jnp.dot(pooled, ew1_ref[...],
                    preferred_element_type=jnp.float32,
                    precision=jax.lax.Precision.HIGHEST) + eb1_ref[...]
        h = _layernorm(h, eg_ref[...], ebeta_ref[...])
        h = _gelu_exact(h)
        comm = jnp.dot(h, ew2_ref[...],
                       preferred_element_type=jnp.float32,
                       precision=jax.lax.Precision.HIGHEST) + eb2_ref[...]
        edges = [edges_ref[i] for i in range(n_edges)]
        centers = [(edges[i] + edges[i + 1]) * 0.5 for i in range(n_edges - 1)]
        s = jnp.zeros_like(comm)
        for i in range(n_edges):
            s += (comm > edges[i]).astype(jnp.float32)
        xq = jnp.zeros_like(comm) + centers[0]
        for i in range(1, n_edges - 1):
            xq += (s > (i + 0.5)).astype(jnp.float32) * (centers[i] - centers[i - 1])
        xq = jnp.where(comm <= edges[0], edges[0], xq)
        xq = jnp.where(comm > edges[-1], edges[-1], xq)
        h2 = jnp.dot(xq, dw1_ref[...],
                     preferred_element_type=jnp.float32,
                     precision=jax.lax.Precision.HIGHEST) + db1_ref[...]
        h2 = _layernorm(h2, dg_ref[...], dbeta_ref[...])
        h2 = _gelu_exact(h2)
        expanded = jnp.dot(h2, dw2_ref[...],
                           preferred_element_type=jnp.float32,
                           precision=jax.lax.Precision.HIGHEST) + db2_ref[...]
        out_ref[...] = expanded[None]


def _add_kernel(x_ref, e_ref, o_ref):
    o_ref[...] = x_ref[...] + 0.1 * e_ref[...]


def kernel(hidden_states, enc_w1, enc_b1, enc_g, enc_beta, enc_w2, enc_b2,
           dec_w1, dec_b1, dec_g, dec_beta, dec_w2, dec_b2, bin_edges,
           interpret=False):
    B, S, H = hidden_states.shape
    CH = 512
    n_chunks = S // CH
    n_edges = bin_edges.shape[0]

    row = lambda v: v.reshape(1, -1)
    full2d = lambda a: pl.BlockSpec(a.shape, lambda b, j: (0, 0))

    expanded = pl.pallas_call(
        functools.partial(_pool_mlp_kernel, n_chunks, n_edges),
        out_shape=jax.ShapeDtypeStruct((B, 1, H), jnp.float32),
        grid=(B, n_chunks),
        in_specs=[
            pl.BlockSpec((1, CH, H), lambda b, j: (b, j, 0)),
            full2d(enc_w1),
            full2d(row(enc_b1)), full2d(row(enc_g)), full2d(row(enc_beta)),
            full2d(enc_w2), full2d(row(enc_b2)),
            full2d(dec_w1),
            full2d(row(dec_b1)), full2d(row(dec_g)), full2d(row(dec_beta)),
            full2d(dec_w2), full2d(row(dec_b2)),
            pl.BlockSpec(memory_space=pltpu.SMEM),
        ],
        out_specs=pl.BlockSpec((1, 1, H), lambda b, j: (b, 0, 0)),
        scratch_shapes=[pltpu.VMEM((1, H), jnp.float32)],
        compiler_params=pltpu.CompilerParams(
            dimension_semantics=("parallel", "arbitrary"),
        ),
        name="pool_mlp",
        interpret=interpret,
    )(hidden_states, enc_w1,
      row(enc_b1), row(enc_g), row(enc_beta), enc_w2, row(enc_b2),
      dec_w1, row(dec_b1), row(dec_g), row(dec_beta), dec_w2, row(dec_b2),
      bin_edges)

    out = pl.pallas_call(
        _add_kernel,
        out_shape=jax.ShapeDtypeStruct((B, S, H), jnp.float32),
        grid=(B, n_chunks),
        in_specs=[
            pl.BlockSpec((1, CH, H), lambda b, j: (b, j, 0)),
            pl.BlockSpec((1, 1, H), lambda b, j: (b, 0, 0)),
        ],
        out_specs=pl.BlockSpec((1, CH, H), lambda b, j: (b, j, 0)),
        compiler_params=pltpu.CompilerParams(
            dimension_semantics=("parallel", "arbitrary"),
        ),
        name="residual_add",
        interpret=interpret,
    )(hidden_states, expanded)

    return out


# speedup vs baseline: 1.1666x; 1.0512x over previous
import functools
import math

import jax
import jax.numpy as jnp
from jax.experimental import pallas as pl
from jax.experimental.pallas import tpu as pltpu

_EPS = 1e-5


def _layernorm(x, g, b):
    mu = x.mean(axis=-1, keepdims=True)
    var = ((x - mu) ** 2).mean(axis=-1, keepdims=True)
    return (x - mu) * jax.lax.rsqrt(var + _EPS) * g + b


def _gelu_exact(x):
    return 0.5 * x * (1.0 + jax.lax.erf(x * (1.0 / math.sqrt(2.0))))


def _pool_mlp_kernel(n_chunks, n_edges,
                     x_ref, ew1_ref, eb1_ref, eg_ref, ebeta_ref, ew2_ref,
                     eb2_ref, dw1_ref, db1_ref, dg_ref, dbeta_ref, dw2_ref,
                     db2_ref, edges_ref, out_ref, acc_ref):
    j = pl.program_id(1)

    @pl.when(j == 0)
    def _():
        acc_ref[...] = jnp.zeros_like(acc_ref)

    acc_ref[...] += jnp.sum(x_ref[0], axis=0, keepdims=True)

    @pl.when(j == n_chunks - 1)
    def _():
        seq = x_ref.shape[1] * n_chunks
        pooled = acc_ref[...] * (1.0 / seq)
        h = jnp.dot(pooled, ew1_ref[...],
                    preferred_element_type=jnp.float32,
                    precision=jax.lax.Precision.HIGHEST) + eb1_ref[...]
        h = _layernorm(h, eg_ref[...], ebeta_ref[...])
        h = _gelu_exact(h)
        comm = jnp.dot(h, ew2_ref[...],
                       preferred_element_type=jnp.float32,
                       precision=jax.lax.Precision.HIGHEST) + eb2_ref[...]
        edges = [edges_ref[i] for i in range(n_edges)]
        centers = [(edges[i] + edges[i + 1]) * 0.5 for i in range(n_edges - 1)]
        s = jnp.zeros_like(comm)
        for i in range(n_edges):
            s += (comm > edges[i]).astype(jnp.float32)
        xq = jnp.zeros_like(comm) + centers[0]
        for i in range(1, n_edges - 1):
            xq += (s > (i + 0.5)).astype(jnp.float32) * (centers[i] - centers[i - 1])
        xq = jnp.where(comm <= edges[0], edges[0], xq)
        xq = jnp.where(comm > edges[-1], edges[-1], xq)
        h2 = jnp.dot(xq, dw1_ref[...],
                     preferred_element_type=jnp.float32,
                     precision=jax.lax.Precision.HIGHEST) + db1_ref[...]
        h2 = _layernorm(h2, dg_ref[...], dbeta_ref[...])
        h2 = _gelu_exact(h2)
        expanded = jnp.dot(h2, dw2_ref[...],
                           preferred_element_type=jnp.float32,
                           precision=jax.lax.Precision.HIGHEST) + db2_ref[...]
        out_ref[...] = expanded[None]


def _add_kernel(x_ref, e_ref, o_ref):
    o_ref[...] = x_ref[...] + 0.1 * e_ref[...]


def kernel(hidden_states, enc_w1, enc_b1, enc_g, enc_beta, enc_w2, enc_b2,
           dec_w1, dec_b1, dec_g, dec_beta, dec_w2, dec_b2, bin_edges,
           interpret=False):
    B, S, H = hidden_states.shape
    CH = 1024
    n_chunks = S // CH
    n_edges = bin_edges.shape[0]

    row = lambda v: v.reshape(1, -1)
    full2d = lambda a: pl.BlockSpec(a.shape, lambda b, j: (0, 0))

    expanded = pl.pallas_call(
        functools.partial(_pool_mlp_kernel, n_chunks, n_edges),
        out_shape=jax.ShapeDtypeStruct((B, 1, H), jnp.float32),
        grid=(B, n_chunks),
        in_specs=[
            pl.BlockSpec((1, CH, H), lambda b, j: (b, j, 0)),
            full2d(enc_w1),
            full2d(row(enc_b1)), full2d(row(enc_g)), full2d(row(enc_beta)),
            full2d(enc_w2), full2d(row(enc_b2)),
            full2d(dec_w1),
            full2d(row(dec_b1)), full2d(row(dec_g)), full2d(row(dec_beta)),
            full2d(dec_w2), full2d(row(dec_b2)),
            pl.BlockSpec(memory_space=pltpu.SMEM),
        ],
        out_specs=pl.BlockSpec((1, 1, H), lambda b, j: (b, 0, 0)),
        scratch_shapes=[pltpu.VMEM((1, H), jnp.float32)],
        compiler_params=pltpu.CompilerParams(
            dimension_semantics=("parallel", "arbitrary"),
            vmem_limit_bytes=56 * 1024 * 1024,
        ),
        name="pool_mlp",
        interpret=interpret,
    )(hidden_states, enc_w1,
      row(enc_b1), row(enc_g), row(enc_beta), enc_w2, row(enc_b2),
      dec_w1, row(dec_b1), row(dec_g), row(dec_beta), dec_w2, row(dec_b2),
      bin_edges)

    out = pl.pallas_call(
        _add_kernel,
        out_shape=jax.ShapeDtypeStruct((B, S, H), jnp.float32),
        grid=(B, n_chunks),
        in_specs=[
            pl.BlockSpec((1, CH, H), lambda b, j: (b, j, 0)),
            pl.BlockSpec((1, 1, H), lambda b, j: (b, 0, 0)),
        ],
        out_specs=pl.BlockSpec((1, CH, H), lambda b, j: (b, j, 0)),
        compiler_params=pltpu.CompilerParams(
            dimension_semantics=("parallel", "arbitrary"),
            vmem_limit_bytes=56 * 1024 * 1024,
        ),
        name="residual_add",
        interpret=interpret,
    )(hidden_states, expanded)

    return out


# speedup vs baseline: 1.1757x; 1.0078x over previous
import functools
import math

import jax
import jax.numpy as jnp
from jax.experimental import pallas as pl
from jax.experimental.pallas import tpu as pltpu

_EPS = 1e-5


def _layernorm(x, g, b):
    mu = x.mean(axis=-1, keepdims=True)
    var = ((x - mu) ** 2).mean(axis=-1, keepdims=True)
    return (x - mu) * jax.lax.rsqrt(var + _EPS) * g + b


def _gelu_exact(x):
    return 0.5 * x * (1.0 + jax.lax.erf(x * (1.0 / math.sqrt(2.0))))


def _mlp_quant_chain(pooled, ew1, eb1, eg, ebeta, ew2, eb2,
                     dw1, db1, dg, dbeta, dw2, db2, edges_ref, n_edges):
    hi = jax.lax.Precision.HIGHEST
    h = jnp.dot(pooled, ew1, preferred_element_type=jnp.float32,
                precision=hi) + eb1
    h = _layernorm(h, eg, ebeta)
    h = _gelu_exact(h)
    comm = jnp.dot(h, ew2, preferred_element_type=jnp.float32,
                   precision=hi) + eb2
    edges = [edges_ref[i] for i in range(n_edges)]
    centers = [(edges[i] + edges[i + 1]) * 0.5 for i in range(n_edges - 1)]
    s = jnp.zeros_like(comm)
    for i in range(n_edges):
        s += (comm > edges[i]).astype(jnp.float32)
    xq = jnp.zeros_like(comm) + centers[0]
    for i in range(1, n_edges - 1):
        xq += (s > (i + 0.5)).astype(jnp.float32) * (centers[i] - centers[i - 1])
    xq = jnp.where(comm <= edges[0], edges[0], xq)
    xq = jnp.where(comm > edges[-1], edges[-1], xq)
    h2 = jnp.dot(xq, dw1, preferred_element_type=jnp.float32,
                 precision=hi) + db1
    h2 = _layernorm(h2, dg, dbeta)
    h2 = _gelu_exact(h2)
    return jnp.dot(h2, dw2, preferred_element_type=jnp.float32,
                   precision=hi) + db2


def _fused_kernel(n_chunks, n_edges,
                  x_ref, ew1_ref, eb1_ref, eg_ref, ebeta_ref, ew2_ref,
                  eb2_ref, dw1_ref, db1_ref, dg_ref, dbeta_ref, dw2_ref,
                  db2_ref, edges_ref, out_ref, acc_ref, evec_ref):
    p = pl.program_id(1)
    j = pl.program_id(2)

    @pl.when((p == 0) & (j == 0))
    def _():
        acc_ref[...] = jnp.zeros_like(acc_ref)

    @pl.when(p == 0)
    def _():
        acc_ref[...] += jnp.sum(x_ref[0], axis=0, keepdims=True)

    @pl.when((p == 1) & (j == 0))
    def _():
        seq = x_ref.shape[1] * n_chunks
        pooled = acc_ref[...] * (1.0 / seq)
        evec_ref[...] = _mlp_quant_chain(
            pooled, ew1_ref[...], eb1_ref[...], eg_ref[...], ebeta_ref[...],
            ew2_ref[...], eb2_ref[...], dw1_ref[...], db1_ref[...],
            dg_ref[...], dbeta_ref[...], dw2_ref[...], db2_ref[...],
            edges_ref, n_edges)

    @pl.when(p == 1)
    def _():
        out_ref[...] = x_ref[...] + 0.1 * evec_ref[...][None]


def kernel(hidden_states, enc_w1, enc_b1, enc_g, enc_beta, enc_w2, enc_b2,
           dec_w1, dec_b1, dec_g, dec_beta, dec_w2, dec_b2, bin_edges,
           interpret=False):
    B, S, H = hidden_states.shape
    CH = 1024
    n_chunks = S // CH
    n_edges = bin_edges.shape[0]

    row = lambda v: v.reshape(1, -1)
    full2d = lambda a: pl.BlockSpec(a.shape, lambda b, p, j: (0, 0))

    out = pl.pallas_call(
        functools.partial(_fused_kernel, n_chunks, n_edges),
        out_shape=jax.ShapeDtypeStruct((B, S, H), jnp.float32),
        grid=(B, 2, n_chunks),
        in_specs=[
            pl.BlockSpec((1, CH, H), lambda b, p, j: (b, j, 0)),
            full2d(enc_w1),
            full2d(row(enc_b1)), full2d(row(enc_g)), full2d(row(enc_beta)),
            full2d(enc_w2), full2d(row(enc_b2)),
            full2d(dec_w1),
            full2d(row(dec_b1)), full2d(row(dec_g)), full2d(row(dec_beta)),
            full2d(dec_w2), full2d(row(dec_b2)),
            pl.BlockSpec(memory_space=pltpu.SMEM),
        ],
        out_specs=pl.BlockSpec(
            (1, CH, H), lambda b, p, j: (b, jnp.where(p == 0, 0, j), 0)),
        scratch_shapes=[
            pltpu.VMEM((1, H), jnp.float32),
            pltpu.VMEM((1, H), jnp.float32),
        ],
        compiler_params=pltpu.CompilerParams(
            dimension_semantics=("parallel", "arbitrary", "arbitrary"),
            vmem_limit_bytes=56 * 1024 * 1024,
        ),
        name="vector_comm_fused",
        interpret=interpret,
    )(hidden_states, enc_w1,
      row(enc_b1), row(enc_g), row(enc_beta), enc_w2, row(enc_b2),
      dec_w1, row(dec_b1), row(dec_g), row(dec_beta), dec_w2, row(dec_b2),
      bin_edges)

    return out


# speedup vs baseline: 1.2142x; 1.0327x over previous
import functools
import math

import jax
import jax.numpy as jnp
from jax.experimental import pallas as pl
from jax.experimental.pallas import tpu as pltpu

_EPS = 1e-5


def _layernorm(x, g, b):
    mu = x.mean(axis=-1, keepdims=True)
    var = ((x - mu) ** 2).mean(axis=-1, keepdims=True)
    return (x - mu) * jax.lax.rsqrt(var + _EPS) * g + b


def _gelu_exact(x):
    return 0.5 * x * (1.0 + jax.lax.erf(x * (1.0 / math.sqrt(2.0))))


def _mlp_quant_chain(pooled, ew1, eb1, eg, ebeta, ew2, eb2,
                     dw1, db1, dg, dbeta, dw2, db2, edges_ref, n_edges):
    hi = jax.lax.Precision.HIGHEST
    h = jnp.dot(pooled, ew1, preferred_element_type=jnp.float32,
                precision=hi) + eb1
    h = _layernorm(h, eg, ebeta)
    h = _gelu_exact(h)
    comm = jnp.dot(h, ew2, preferred_element_type=jnp.float32,
                   precision=hi) + eb2
    edges = [edges_ref[i] for i in range(n_edges)]
    centers = [(edges[i] + edges[i + 1]) * 0.5 for i in range(n_edges - 1)]
    s = jnp.zeros_like(comm)
    for i in range(n_edges):
        s += (comm > edges[i]).astype(jnp.float32)
    xq = jnp.zeros_like(comm) + centers[0]
    for i in range(1, n_edges - 1):
        xq += (s > (i + 0.5)).astype(jnp.float32) * (centers[i] - centers[i - 1])
    xq = jnp.where(comm <= edges[0], edges[0], xq)
    xq = jnp.where(comm > edges[-1], edges[-1], xq)
    h2 = jnp.dot(xq, dw1, preferred_element_type=jnp.float32,
                 precision=hi) + db1
    h2 = _layernorm(h2, dg, dbeta)
    h2 = _gelu_exact(h2)
    return jnp.dot(h2, dw2, preferred_element_type=jnp.float32,
                   precision=hi) + db2


def _fused_kernel(n_batches, n_chunks, n_edges,
                  xa_ref, xb_ref, ew1_ref, eb1_ref, eg_ref, ebeta_ref,
                  ew2_ref, eb2_ref, dw1_ref, db1_ref, dg_ref, dbeta_ref,
                  dw2_ref, db2_ref, edges_ref, out_ref, acc_ref, evec_ref):
    k = pl.program_id(0)
    j = pl.program_id(1)

    @pl.when((k == 0) & (j == 0))
    def _():
        acc_ref[...] = jnp.zeros_like(acc_ref)

    @pl.when(k < n_batches)
    def _():
        kc = jnp.minimum(k, n_batches - 1)
        acc_ref[pl.ds(kc, 1), :] += jnp.sum(xb_ref[0], axis=0, keepdims=True)

    @pl.when((k >= 1) & (j == 0))
    def _():
        seq = xb_ref.shape[1] * n_chunks
        kp = jnp.maximum(k - 1, 0)
        pooled = acc_ref[pl.ds(kp, 1), :] * (1.0 / seq)
        evec_ref[...] = _mlp_quant_chain(
            pooled, ew1_ref[...], eb1_ref[...], eg_ref[...], ebeta_ref[...],
            ew2_ref[...], eb2_ref[...], dw1_ref[...], db1_ref[...],
            dg_ref[...], dbeta_ref[...], dw2_ref[...], db2_ref[...],
            edges_ref, n_edges)

    @pl.when(k >= 1)
    def _():
        out_ref[...] = xa_ref[...] + 0.1 * evec_ref[...][None]


def kernel(hidden_states, enc_w1, enc_b1, enc_g, enc_beta, enc_w2, enc_b2,
           dec_w1, dec_b1, dec_g, dec_beta, dec_w2, dec_b2, bin_edges,
           interpret=False):
    B, S, H = hidden_states.shape
    CH = 1024
    n_chunks = S // CH
    n_edges = bin_edges.shape[0]

    row = lambda v: v.reshape(1, -1)
    full2d = lambda a: pl.BlockSpec(a.shape, lambda k, j: (0, 0))

    xa_spec = pl.BlockSpec(
        (1, CH, H),
        lambda k, j: (jnp.maximum(k - 1, 0), jnp.where(k == 0, 0, j), 0))
    xb_spec = pl.BlockSpec(
        (1, CH, H),
        lambda k, j: (jnp.minimum(k, B - 1),
                      jnp.where(k >= B, n_chunks - 1, j), 0))
    out_spec = pl.BlockSpec(
        (1, CH, H),
        lambda k, j: (jnp.maximum(k - 1, 0), jnp.where(k == 0, 0, j), 0))

    out = pl.pallas_call(
        functools.partial(_fused_kernel, B, n_chunks, n_edges),
        out_shape=jax.ShapeDtypeStruct((B, S, H), jnp.float32),
        grid=(B + 1, n_chunks),
        in_specs=[
            xa_spec, xb_spec,
            full2d(enc_w1),
            full2d(row(enc_b1)), full2d(row(enc_g)), full2d(row(enc_beta)),
            full2d(enc_w2), full2d(row(enc_b2)),
            full2d(dec_w1),
            full2d(row(dec_b1)), full2d(row(dec_g)), full2d(row(dec_beta)),
            full2d(dec_w2), full2d(row(dec_b2)),
            pl.BlockSpec(memory_space=pltpu.SMEM),
        ],
        out_specs=out_spec,
        scratch_shapes=[
            pltpu.VMEM((B, H), jnp.float32),
            pltpu.VMEM((1, H), jnp.float32),
        ],
        compiler_params=pltpu.CompilerParams(
            dimension_semantics=("arbitrary", "arbitrary"),
            vmem_limit_bytes=56 * 1024 * 1024,
        ),
        name="vector_comm_fused",
        interpret=interpret,
    )(hidden_states, hidden_states, enc_w1,
      row(enc_b1), row(enc_g), row(enc_beta), enc_w2, row(enc_b2),
      dec_w1, row(dec_b1), row(dec_g), row(dec_beta), dec_w2, row(dec_b2),
      bin_edges)

    return out
